# Initial kernel scaffold; baseline (speedup 1.0000x reference)
import jax
import jax.numpy as jnp
from jax.experimental import pallas as pl


def kernel(x):
    raise NotImplementedError("write your pallas kernel here")


# speedup vs baseline: 6.2094x; 6.2094x over previous
import functools

import jax
import jax.numpy as jnp
from jax.experimental import pallas as pl
from jax.experimental.pallas import tpu as pltpu

K = 64
ROWS_PER_BLOCK = 8
CHUNKS = 256
LANES = 128


def _topk_mask_kernel(x_ref, o_ref):
    x = x_ref[...]
    bits = jax.lax.bitcast_convert_type(x, jnp.int32) & jnp.int32(0x7FFFFFFF)

    def count_ge(t):
        return jnp.sum((bits >= t).astype(jnp.int32), axis=(1, 2), keepdims=True)

    def body(_, carry):
        lo, hi = carry
        mid = lo + (hi - lo) // 2
        c = count_ge(mid)
        big = c >= K
        return jnp.where(big, mid, lo), jnp.where(big, hi, mid)

    r = x.shape[0]
    lo0 = jnp.zeros((r, 1, 1), jnp.int32)
    hi0 = jnp.full((r, 1, 1), 0x7F800000, jnp.int32)
    lo, hi = jax.lax.fori_loop(0, 31, body, (lo0, hi0), unroll=True)
    gt = bits > lo
    eq = bits == lo
    n_gt = jnp.sum(gt.astype(jnp.int32), axis=(1, 2), keepdims=True)
    need = K - n_gt

    eqf = eq.astype(jnp.float32)
    lane_tri = (jax.lax.broadcasted_iota(jnp.int32, (LANES, LANES), 0)
                < jax.lax.broadcasted_iota(jnp.int32, (LANES, LANES), 1)
                ).astype(jnp.float32)
    lane_pre = jax.lax.dot_general(
        eqf, lane_tri, (((2,), (0,)), ((), ())),
        preferred_element_type=jnp.float32)
    chunk_tot = jnp.sum(eqf, axis=2)
    chunk_tri = (jax.lax.broadcasted_iota(jnp.int32, (CHUNKS, CHUNKS), 0)
                 < jax.lax.broadcasted_iota(jnp.int32, (CHUNKS, CHUNKS), 1)
                 ).astype(jnp.float32)
    chunk_pre = jax.lax.dot_general(
        chunk_tot, chunk_tri, (((1,), (0,)), ((), ())),
        preferred_element_type=jnp.float32)
    prefix = lane_pre + chunk_pre[:, :, None]

    keep = gt | (eq & (prefix < need.astype(jnp.float32)))
    o_ref[...] = jnp.where(keep, x, 0.0)


@jax.jit
def kernel(x):
    n_rows, n = x.shape
    x3 = x.reshape(n_rows, CHUNKS, LANES)
    grid = (n_rows // ROWS_PER_BLOCK,)
    out = pl.pallas_call(
        _topk_mask_kernel,
        out_shape=jax.ShapeDtypeStruct(x3.shape, x3.dtype),
        grid=grid,
        in_specs=[pl.BlockSpec((ROWS_PER_BLOCK, CHUNKS, LANES),
                               lambda i: (i, 0, 0))],
        out_specs=pl.BlockSpec((ROWS_PER_BLOCK, CHUNKS, LANES),
                               lambda i: (i, 0, 0)),
        compiler_params=pltpu.CompilerParams(
            dimension_semantics=("parallel",)),
    )(x3)
    return out.reshape(n_rows, n)


# speedup vs baseline: 12.3095x; 1.9824x over previous
import jax
import jax.numpy as jnp
from jax import lax
from jax.experimental import pallas as pl
from jax.experimental.pallas import tpu as pltpu
from jax.experimental.pallas import tpu_sc as plsc

K = 64
N = 32768
ROWS = 128
NV = N // 16
NWORKERS = 32
ROWS_PER_W = ROWS // NWORKERS
H1, H2, H3 = 2048, 1024, 1024


def _bits_of(v):
    return lax.bitcast_convert_type(v, jnp.int32) & jnp.int32(0x7FFFFFFF)


def _find_bucket(hist_ref, nbuckets, target):
    nv = nbuckets // 16

    def body_a(i, c):
        run, vi, run_before = c
        idx = nv - 1 - i
        v = hist_ref[pl.ds(idx * 16, 16)]
        tot = jnp.sum(v)
        newrun = run + tot
        hit = (newrun >= target) & (run < target)
        vi = jnp.where(hit, idx, vi)
        run_before = jnp.where(hit, run, run_before)
        return newrun, vi, run_before

    _, vi, run_before = lax.fori_loop(
        0, nv, body_a, (jnp.int32(0), jnp.int32(0), jnp.int32(0)))
    v = hist_ref[pl.ds(vi * 16, 16)]
    suf = lax.rev(plsc.cumsum(lax.rev(v, (0,))), (0,))
    mask = (run_before + suf) >= target
    jstar = jnp.sum(mask.astype(jnp.int32)) - 1
    sel = jnp.arange(16, dtype=jnp.int32) == jstar
    vj = jnp.sum(jnp.where(sel, v, 0))
    sufj = jnp.sum(jnp.where(sel, suf, 0))
    return vi * 16 + jstar, run_before + sufj - vj, vj


def _do_row(x_hbm, out_hbm, xv, h1, h2, h3, row):
    pltpu.sync_copy(x_hbm.at[row], xv)

    zeros16 = jnp.zeros((16,), jnp.int32)
    ones16 = jnp.ones((16,), jnp.int32)

    @plsc.parallel_loop(0, H1 // 16, unroll=8)
    def _(i):
        h1[pl.ds(i * 16, 16)] = zeros16

    @plsc.parallel_loop(0, H2 // 16, unroll=8)
    def _(i):
        h2[pl.ds(i * 16, 16)] = zeros16
        h3[pl.ds(i * 16, 16)] = zeros16

    @plsc.parallel_loop(0, NV, unroll=8)
    def _(i):
        bits = _bits_of(xv[pl.ds(i * 16, 16)])
        plsc.addupdate_scatter(h1, [bits >> 20], ones16)

    b1, above1, _ = _find_bucket(h1, H1, K)
    r2 = K - above1

    @plsc.parallel_loop(0, NV, unroll=8)
    def _(i):
        bits = _bits_of(xv[pl.ds(i * 16, 16)])
        plsc.addupdate_scatter(h2, [(bits >> 10) & 0x3FF], ones16,
                               mask=(bits >> 20) == b1)

    b2, above2, _ = _find_bucket(h2, H2, r2)
    r3 = r2 - above2
    hi21 = (b1 << 10) | b2

    @plsc.parallel_loop(0, NV, unroll=8)
    def _(i):
        bits = _bits_of(xv[pl.ds(i * 16, 16)])
        plsc.addupdate_scatter(h3, [bits & 0x3FF], ones16,
                               mask=(bits >> 10) == hi21)

    b3, above3, meq = _find_bucket(h3, H3, r3)
    t = (hi21 << 10) | b3
    r = r3 - above3

    def simple(_):
        @plsc.parallel_loop(0, NV, unroll=8)
        def _(i):
            v = xv[pl.ds(i * 16, 16)]
            keep = _bits_of(v) >= t
            xv[pl.ds(i * 16, 16)] = jnp.where(keep, v, 0.0)
        return 0

    def careful(_):
        def body(i, run):
            v = xv[pl.ds(i * 16, 16)]
            bits = _bits_of(v)
            eq = bits == t
            eqi = eq.astype(jnp.int32)
            excl = plsc.cumsum(eqi) - eqi
            keep = (bits > t) | (eq & ((excl + run) < r))
            xv[pl.ds(i * 16, 16)] = jnp.where(keep, v, 0.0)
            return run + jnp.sum(eqi)
        lax.fori_loop(0, NV, body, jnp.int32(0))
        return 0

    lax.cond(r == meq, simple, careful, 0)
    pltpu.sync_copy(xv, out_hbm.at[row])


def _sc_topk(x_hbm, out_hbm, xv, h1, h2, h3):
    wid = lax.axis_index("s") * 2 + lax.axis_index("c")

    def rb(j, _):
        _do_row(x_hbm, out_hbm, xv, h1, h2, h3, wid * ROWS_PER_W + j)
        return 0

    lax.fori_loop(0, ROWS_PER_W, rb, 0)


@jax.jit
def kernel(x):
    f = pl.kernel(
        _sc_topk,
        out_type=jax.ShapeDtypeStruct((ROWS, N), jnp.float32),
        mesh=plsc.VectorSubcoreMesh(core_axis_name="c", subcore_axis_name="s",
                                    num_cores=2, num_subcores=16),
        scratch_types=[
            pltpu.VMEM((N,), jnp.float32),
            pltpu.VMEM((H1,), jnp.int32),
            pltpu.VMEM((H2,), jnp.int32),
            pltpu.VMEM((H3,), jnp.int32),
        ],
        compiler_params=pltpu.CompilerParams(needs_layout_passes=False),
    )
    return f(x)
